# baseline (speedup 1.0000x reference)
import jax
import jax.numpy as jnp
from jax import lax
from jax.experimental import pallas as pl

_N = 65536
_C = 80
_ROWS_A = 2048
_CH = 4096


def _pass_a(yp_ref, yt_ref, mask_ref, sl_ref):
    p = yp_ref[...]
    t = yt_ref[...]
    log_p = jnp.maximum(jnp.log(p), -100.0)
    log_1p = jnp.maximum(jnp.log(1.0 - p), -100.0)
    l = -(t * log_p + (1.0 - t) * log_1p)
    loss = jnp.sum(l, axis=1, keepdims=True)
    m = mask_ref[...]
    sel = (m > 0.5) & (t > 0.5)
    sl_ref[...] = jnp.where(sel, loss, -1.0).astype(jnp.bfloat16)


def _rev_cumsum(h, nbins):
    x = h
    s = 1
    while s < nbins:
        shifted = jnp.concatenate(
            [x[:, s:], jnp.zeros((_C, s), jnp.float32)], axis=1)
        x = x + shifted
        s *= 2
    return x


def _pick_bin(t_cum, m_res, nbins):
    ge = (t_cum >= m_res).astype(jnp.float32)
    b = jnp.sum(ge, axis=1, keepdims=True) - 1.0
    return b.astype(jnp.int32)


def _lane_pick(arr, idx, nbins):
    qi = lax.broadcasted_iota(jnp.int32, (_C, nbins), 1)
    sel = qi == idx
    return jnp.sum(jnp.where(sel, arr, 0.0), axis=1, keepdims=True)


def _pass_b(sl_ref, mask_ref, out_ref):
    nch = _N // _CH
    dn = (((0,), (0,)), ((), ()))

    def _hist_l1(j, acc):
        ch = sl_ref[pl.ds(j * _CH, _CH), :]
        rmax = jnp.max(ch, axis=1, keepdims=True)
        bits = lax.bitcast_convert_type(rmax, jnp.int16).astype(jnp.int32)
        bin1 = bits >> 7
        onehot = (bin1 == lax.broadcasted_iota(
            jnp.int32, (_CH, 256), 1)).astype(jnp.bfloat16)
        msel = (ch >= 0).astype(jnp.bfloat16)
        return acc + lax.dot_general(msel, onehot, dn,
                                     preferred_element_type=jnp.float32)

    h1 = lax.fori_loop(0, nch, _hist_l1, jnp.zeros((_C, 256), jnp.float32))

    ts = jnp.sum(mask_ref[...])
    k = ts.astype(jnp.int32) // 2
    t1 = _rev_cumsum(h1, 256)
    cnt = t1[:, 0:1]
    m = jnp.minimum(cnt.astype(jnp.int32), k)
    m_f = m.astype(jnp.float32)

    b1 = _pick_bin(t1, m_f, 256)
    t1_b = _lane_pick(t1, b1, 256)
    h1_b = _lane_pick(h1, b1, 256)
    m2 = m_f - (t1_b - h1_b)

    def _hist_l2(j, acc):
        ch = sl_ref[pl.ds(j * _CH, _CH), :]
        rmax = jnp.max(ch, axis=1, keepdims=True)
        bits = lax.bitcast_convert_type(rmax, jnp.int16).astype(jnp.int32)
        bin1 = bits >> 7
        bin2 = bits & 127
        onehot = (bin2 == lax.broadcasted_iota(
            jnp.int32, (_CH, 128), 1)).astype(jnp.bfloat16)
        inb1 = bin1 == jnp.reshape(b1, (1, _C))
        msel = ((ch >= 0) & inb1).astype(jnp.bfloat16)
        return acc + lax.dot_general(msel, onehot, dn,
                                     preferred_element_type=jnp.float32)

    h2 = lax.fori_loop(0, nch, _hist_l2, jnp.zeros((_C, 128), jnp.float32))
    t2 = _rev_cumsum(h2, 128)
    b2 = _pick_bin(t2, m2, 128)

    tbits = (b1 << 7) | b2
    tval = lax.bitcast_convert_type(
        tbits.astype(jnp.int16), jnp.bfloat16).astype(jnp.float32)
    t_row = jnp.reshape(tval, (1, _C))

    def _final(j, carry):
        s_acc, c_acc = carry
        ch = sl_ref[pl.ds(j * _CH, _CH), :].astype(jnp.float32)
        gt = ch > t_row
        sv = jnp.where(gt, ch, 0.0)
        cv = gt.astype(jnp.float32)
        return (s_acc + jnp.sum(sv, axis=0, keepdims=True),
                c_acc + jnp.sum(cv, axis=0, keepdims=True))

    z = jnp.zeros((1, _C), jnp.float32)
    s_sum, c_cnt = lax.fori_loop(0, nch, _final, (z, z))

    m_row = jnp.reshape(m_f, (1, _C))
    class_sum = s_sum + (m_row - c_cnt) * t_row
    class_loss = class_sum / m_row
    valid = jnp.reshape(cnt, (1, _C)) > 0.0
    num_valid = jnp.sum(valid.astype(jnp.float32))
    mean_valid = jnp.sum(jnp.where(valid, class_loss, 0.0)) / num_valid
    result = jnp.where(ts > 0.0, mean_valid, 0.0)
    out_ref[...] = result * jnp.ones((1, 1), jnp.float32)


def kernel(y_pred, y_true, mask):
    n, c = y_pred.shape
    mask_f = mask.astype(jnp.float32).reshape(n, 1)

    sel_loss = pl.pallas_call(
        _pass_a,
        grid=(n // _ROWS_A,),
        in_specs=[
            pl.BlockSpec((_ROWS_A, c), lambda i: (i, 0)),
            pl.BlockSpec((_ROWS_A, c), lambda i: (i, 0)),
            pl.BlockSpec((_ROWS_A, 1), lambda i: (i, 0)),
        ],
        out_specs=pl.BlockSpec((_ROWS_A, c), lambda i: (i, 0)),
        out_shape=jax.ShapeDtypeStruct((n, c), jnp.bfloat16),
    )(y_pred, y_true, mask_f)

    mask_rm = mask_f.reshape(n // 128, 128)

    out = pl.pallas_call(
        _pass_b,
        out_shape=jax.ShapeDtypeStruct((1, 1), jnp.float32),
    )(sel_loss, mask_rm)

    return out[0, 0]


# speedup vs baseline: 1.0144x; 1.0144x over previous
import jax
import jax.numpy as jnp
from jax import lax
from jax.experimental import pallas as pl
from jax.experimental.pallas import tpu as pltpu

_N = 65536
_C = 80
_ROWS_A = 2048
_CH = 4096


def _pass_a(yp_ref, yt_ref, mask_ref, sl_ref, h1_ref, w1_ref):
    p = yp_ref[...]
    t = yt_ref[...]
    log_p = jnp.maximum(jnp.log(p), -100.0)
    log_1p = jnp.maximum(jnp.log(1.0 - p), -100.0)
    l = -(t * log_p + (1.0 - t) * log_1p)
    loss = jnp.sum(l, axis=1, keepdims=True)
    mk = mask_ref[...]
    sel = (mk > 0.5) & (t > 0.5)
    slf = jnp.where(sel, loss, -1.0)
    slb = slf.astype(jnp.bfloat16)
    sl_ref[...] = slb

    loss_bf = loss.astype(jnp.bfloat16)
    bits = lax.bitcast_convert_type(loss_bf, jnp.int16).astype(jnp.int32)
    bin1 = bits >> 7
    onehot = (bin1 == lax.broadcasted_iota(
        jnp.int32, (_ROWS_A, 256), 1)).astype(jnp.bfloat16)
    mselb = (slb >= 0).astype(jnp.bfloat16)
    wselb = jnp.maximum(slb, jnp.bfloat16(0.0))

    dn = (((0,), (0,)), ((), ()))

    @pl.when(pl.program_id(0) == 0)
    def _():
        h1_ref[...] = jnp.zeros((_C, 256), jnp.float32)
        w1_ref[...] = jnp.zeros((_C, 256), jnp.float32)

    h1_ref[...] += lax.dot_general(mselb, onehot, dn,
                                   preferred_element_type=jnp.float32)
    w1_ref[...] += lax.dot_general(wselb, onehot, dn,
                                   preferred_element_type=jnp.float32)


def _rev_cumsum(h, nbins):
    x = h
    s = 1
    while s < nbins:
        shifted = jnp.concatenate(
            [x[:, s:], jnp.zeros((_C, s), jnp.float32)], axis=1)
        x = x + shifted
        s *= 2
    return x


def _pick_bin(t_cum, m_res, nbins):
    ge = (t_cum >= m_res).astype(jnp.float32)
    b = jnp.sum(ge, axis=1, keepdims=True) - 1.0
    return b.astype(jnp.int32)


def _lane_pick(arr, idx, nbins):
    qi = lax.broadcasted_iota(jnp.int32, (_C, nbins), 1)
    pick = qi == idx
    return jnp.sum(jnp.where(pick, arr, 0.0), axis=1, keepdims=True)


def _pass_b(sl_ref, h1_ref, w1_ref, mask_ref, out_ref, h2_ref, w2_ref):
    j = pl.program_id(0)
    nsteps = pl.num_programs(0)

    ts = jnp.sum(mask_ref[...])
    k = ts.astype(jnp.int32) // 2
    h1 = h1_ref[...]
    t1 = _rev_cumsum(h1, 256)
    cnt = t1[:, 0:1]
    m = jnp.minimum(cnt.astype(jnp.int32), k)
    m_f = m.astype(jnp.float32)
    b1 = _pick_bin(t1, m_f, 256)
    t1_b = _lane_pick(t1, b1, 256)
    h1_b = _lane_pick(h1, b1, 256)
    m2 = m_f - (t1_b - h1_b)

    ch = sl_ref[...]
    rmax = jnp.max(ch, axis=1, keepdims=True)
    bits = lax.bitcast_convert_type(rmax, jnp.int16).astype(jnp.int32)
    bin1r = bits >> 7
    bin2r = bits & 127
    inb1 = bin1r == jnp.reshape(b1, (1, _C))
    selm = ch >= 0
    keep = selm & inb1
    msel2 = keep.astype(jnp.bfloat16)
    wsel2 = jnp.where(keep, ch, jnp.bfloat16(0.0))
    oh2 = (bin2r == lax.broadcasted_iota(
        jnp.int32, (_CH, 128), 1)).astype(jnp.bfloat16)

    dn = (((0,), (0,)), ((), ()))

    @pl.when(j == 0)
    def _():
        h2_ref[...] = jnp.zeros((_C, 128), jnp.float32)
        w2_ref[...] = jnp.zeros((_C, 128), jnp.float32)

    h2_ref[...] += lax.dot_general(msel2, oh2, dn,
                                   preferred_element_type=jnp.float32)
    w2_ref[...] += lax.dot_general(wsel2, oh2, dn,
                                   preferred_element_type=jnp.float32)

    @pl.when(j == nsteps - 1)
    def _():
        h2 = h2_ref[...]
        w2 = w2_ref[...]
        t2 = _rev_cumsum(h2, 128)
        b2 = _pick_bin(t2, m2, 128)
        t2_b = _lane_pick(t2, b2, 128)
        h2_b = _lane_pick(h2, b2, 128)

        tbits = (b1 << 7) | b2
        tval = lax.bitcast_convert_type(
            tbits.astype(jnp.int16), jnp.bfloat16).astype(jnp.float32)

        w1 = w1_ref[...]
        wr1 = _rev_cumsum(w1, 256)
        w1_above = _lane_pick(wr1, b1, 256) - _lane_pick(w1, b1, 256)
        wr2 = _rev_cumsum(w2, 128)
        w2_above = _lane_pick(wr2, b2, 128) - _lane_pick(w2, b2, 128)

        s_gt = w1_above + w2_above
        c_gt = (t1_b - h1_b) + (t2_b - h2_b)

        class_sum = s_gt + (m_f - c_gt) * tval
        class_loss = class_sum / m_f
        valid = cnt > 0.0
        num_valid = jnp.sum(valid.astype(jnp.float32))
        mean_valid = jnp.sum(jnp.where(valid, class_loss, 0.0)) / num_valid
        result = jnp.where(ts > 0.0, mean_valid, 0.0)
        out_ref[...] = result * jnp.ones((1, 1), jnp.float32)


def kernel(y_pred, y_true, mask):
    n, c = y_pred.shape
    mask_f = mask.astype(jnp.float32).reshape(n, 1)

    sel_loss, h1, w1 = pl.pallas_call(
        _pass_a,
        grid=(n // _ROWS_A,),
        in_specs=[
            pl.BlockSpec((_ROWS_A, c), lambda i: (i, 0)),
            pl.BlockSpec((_ROWS_A, c), lambda i: (i, 0)),
            pl.BlockSpec((_ROWS_A, 1), lambda i: (i, 0)),
        ],
        out_specs=[
            pl.BlockSpec((_ROWS_A, c), lambda i: (i, 0)),
            pl.BlockSpec((_C, 256), lambda i: (0, 0)),
            pl.BlockSpec((_C, 256), lambda i: (0, 0)),
        ],
        out_shape=[
            jax.ShapeDtypeStruct((n, c), jnp.bfloat16),
            jax.ShapeDtypeStruct((_C, 256), jnp.float32),
            jax.ShapeDtypeStruct((_C, 256), jnp.float32),
        ],
    )(y_pred, y_true, mask_f)

    mask_rm = mask_f.reshape(n // 128, 128)

    out = pl.pallas_call(
        _pass_b,
        grid=(n // _CH,),
        in_specs=[
            pl.BlockSpec((_CH, c), lambda j: (j, 0)),
            pl.BlockSpec((_C, 256), lambda j: (0, 0)),
            pl.BlockSpec((_C, 256), lambda j: (0, 0)),
            pl.BlockSpec((n // 128, 128), lambda j: (0, 0)),
        ],
        out_specs=pl.BlockSpec((1, 1), lambda j: (0, 0)),
        out_shape=jax.ShapeDtypeStruct((1, 1), jnp.float32),
        scratch_shapes=[
            pltpu.VMEM((_C, 128), jnp.float32),
            pltpu.VMEM((_C, 128), jnp.float32),
        ],
    )(sel_loss, h1, w1, mask_rm)

    return out[0, 0]


# speedup vs baseline: 1.1429x; 1.1267x over previous
import jax
import jax.numpy as jnp
from jax import lax
from jax.experimental import pallas as pl
from jax.experimental.pallas import tpu as pltpu

_N = 65536
_C = 80
_ROWS_A = 4096
_CH = 8192


def _pass_a(yp_ref, yt_ref, mask_ref, sl_ref, h1_ref, w1_ref):
    p = yp_ref[...]
    t = yt_ref[...]
    log_p = jnp.maximum(jnp.log(p), -100.0)
    log_1p = jnp.maximum(jnp.log(1.0 - p), -100.0)
    l = -(t * log_p + (1.0 - t) * log_1p)
    loss = jnp.sum(l, axis=1, keepdims=True)
    mk = mask_ref[...]
    sel = (mk > 0.5) & (t > 0.5)
    slf = jnp.where(sel, loss, -1.0)
    slb = slf.astype(jnp.bfloat16)
    sl_ref[...] = slb

    loss_bf = loss.astype(jnp.bfloat16)
    bits = lax.bitcast_convert_type(loss_bf, jnp.int16).astype(jnp.int32)
    bin1 = bits >> 7
    onehot = (bin1 == lax.broadcasted_iota(
        jnp.int32, (_ROWS_A, 256), 1)).astype(jnp.bfloat16)
    mselb = (slb >= 0).astype(jnp.bfloat16)
    wselb = jnp.maximum(slb, jnp.bfloat16(0.0))

    dn = (((0,), (0,)), ((), ()))

    @pl.when(pl.program_id(0) == 0)
    def _():
        h1_ref[...] = jnp.zeros((_C, 256), jnp.float32)
        w1_ref[...] = jnp.zeros((_C, 256), jnp.float32)

    h1_ref[...] += lax.dot_general(mselb, onehot, dn,
                                   preferred_element_type=jnp.float32)
    w1_ref[...] += lax.dot_general(wselb, onehot, dn,
                                   preferred_element_type=jnp.float32)


def _rev_cumsum(h, nbins):
    x = h
    s = 1
    while s < nbins:
        shifted = jnp.concatenate(
            [x[:, s:], jnp.zeros((_C, s), jnp.float32)], axis=1)
        x = x + shifted
        s *= 2
    return x


def _pick_bin(t_cum, m_res, nbins):
    ge = (t_cum >= m_res).astype(jnp.float32)
    b = jnp.sum(ge, axis=1, keepdims=True) - 1.0
    return b.astype(jnp.int32)


def _lane_pick(arr, idx, nbins):
    qi = lax.broadcasted_iota(jnp.int32, (_C, nbins), 1)
    pick = qi == idx
    return jnp.sum(jnp.where(pick, arr, 0.0), axis=1, keepdims=True)


def _pass_b(sl_ref, h1_ref, w1_ref, mask_ref, out_ref, h2_ref, w2_ref):
    j = pl.program_id(0)
    nsteps = pl.num_programs(0)

    ts = jnp.sum(mask_ref[...])
    k = ts.astype(jnp.int32) // 2
    h1 = h1_ref[...]
    t1 = _rev_cumsum(h1, 256)
    cnt = t1[:, 0:1]
    m = jnp.minimum(cnt.astype(jnp.int32), k)
    m_f = m.astype(jnp.float32)
    b1 = _pick_bin(t1, m_f, 256)
    t1_b = _lane_pick(t1, b1, 256)
    h1_b = _lane_pick(h1, b1, 256)
    m2 = m_f - (t1_b - h1_b)

    ch = sl_ref[...]
    rmax = jnp.max(ch, axis=1, keepdims=True)
    bits = lax.bitcast_convert_type(rmax, jnp.int16).astype(jnp.int32)
    bin1r = bits >> 7
    bin2r = bits & 127
    inb1 = bin1r == jnp.reshape(b1, (1, _C))
    selm = ch >= 0
    keep = selm & inb1
    msel2 = keep.astype(jnp.bfloat16)
    wsel2 = jnp.where(keep, ch, jnp.bfloat16(0.0))
    oh2 = (bin2r == lax.broadcasted_iota(
        jnp.int32, (_CH, 128), 1)).astype(jnp.bfloat16)

    dn = (((0,), (0,)), ((), ()))

    @pl.when(j == 0)
    def _():
        h2_ref[...] = jnp.zeros((_C, 128), jnp.float32)
        w2_ref[...] = jnp.zeros((_C, 128), jnp.float32)

    h2_ref[...] += lax.dot_general(msel2, oh2, dn,
                                   preferred_element_type=jnp.float32)
    w2_ref[...] += lax.dot_general(wsel2, oh2, dn,
                                   preferred_element_type=jnp.float32)

    @pl.when(j == nsteps - 1)
    def _():
        h2 = h2_ref[...]
        w2 = w2_ref[...]
        t2 = _rev_cumsum(h2, 128)
        b2 = _pick_bin(t2, m2, 128)
        t2_b = _lane_pick(t2, b2, 128)
        h2_b = _lane_pick(h2, b2, 128)

        tbits = (b1 << 7) | b2
        tval = lax.bitcast_convert_type(
            tbits.astype(jnp.int16), jnp.bfloat16).astype(jnp.float32)

        w1 = w1_ref[...]
        wr1 = _rev_cumsum(w1, 256)
        w1_above = _lane_pick(wr1, b1, 256) - _lane_pick(w1, b1, 256)
        wr2 = _rev_cumsum(w2, 128)
        w2_above = _lane_pick(wr2, b2, 128) - _lane_pick(w2, b2, 128)

        s_gt = w1_above + w2_above
        c_gt = (t1_b - h1_b) + (t2_b - h2_b)

        class_sum = s_gt + (m_f - c_gt) * tval
        class_loss = class_sum / m_f
        valid = cnt > 0.0
        num_valid = jnp.sum(valid.astype(jnp.float32))
        mean_valid = jnp.sum(jnp.where(valid, class_loss, 0.0)) / num_valid
        result = jnp.where(ts > 0.0, mean_valid, 0.0)
        out_ref[...] = result * jnp.ones((1, 1), jnp.float32)


def kernel(y_pred, y_true, mask):
    n, c = y_pred.shape
    mask_f = mask.astype(jnp.float32).reshape(n, 1)

    sel_loss, h1, w1 = pl.pallas_call(
        _pass_a,
        grid=(n // _ROWS_A,),
        in_specs=[
            pl.BlockSpec((_ROWS_A, c), lambda i: (i, 0)),
            pl.BlockSpec((_ROWS_A, c), lambda i: (i, 0)),
            pl.BlockSpec((_ROWS_A, 1), lambda i: (i, 0)),
        ],
        out_specs=[
            pl.BlockSpec((_ROWS_A, c), lambda i: (i, 0)),
            pl.BlockSpec((_C, 256), lambda i: (0, 0)),
            pl.BlockSpec((_C, 256), lambda i: (0, 0)),
        ],
        out_shape=[
            jax.ShapeDtypeStruct((n, c), jnp.bfloat16),
            jax.ShapeDtypeStruct((_C, 256), jnp.float32),
            jax.ShapeDtypeStruct((_C, 256), jnp.float32),
        ],
    )(y_pred, y_true, mask_f)

    mask_rm = mask_f.reshape(n // 128, 128)

    out = pl.pallas_call(
        _pass_b,
        grid=(n // _CH,),
        in_specs=[
            pl.BlockSpec((_CH, c), lambda j: (j, 0)),
            pl.BlockSpec((_C, 256), lambda j: (0, 0)),
            pl.BlockSpec((_C, 256), lambda j: (0, 0)),
            pl.BlockSpec((n // 128, 128), lambda j: (0, 0)),
        ],
        out_specs=pl.BlockSpec((1, 1), lambda j: (0, 0)),
        out_shape=jax.ShapeDtypeStruct((1, 1), jnp.float32),
        scratch_shapes=[
            pltpu.VMEM((_C, 128), jnp.float32),
            pltpu.VMEM((_C, 128), jnp.float32),
        ],
    )(sel_loss, h1, w1, mask_rm)

    return out[0, 0]


# speedup vs baseline: 1.1830x; 1.0351x over previous
import jax
import jax.numpy as jnp
from jax import lax
from jax.experimental import pallas as pl
from jax.experimental.pallas import tpu as pltpu

_N = 65536
_C = 80
_ROWS_A = 8192
_CH = 16384


def _pass_a(yp_ref, yt_ref, mask_ref, sl_ref, h1_ref, w1_ref):
    p = yp_ref[...]
    t = yt_ref[...]
    log_p = jnp.maximum(jnp.log(p), -100.0)
    log_1p = jnp.maximum(jnp.log(1.0 - p), -100.0)
    l = -(t * log_p + (1.0 - t) * log_1p)
    loss = jnp.sum(l, axis=1, keepdims=True)
    mk = mask_ref[...]
    sel = (mk > 0.5) & (t > 0.5)
    slf = jnp.where(sel, loss, -1.0)
    slb = slf.astype(jnp.bfloat16)
    sl_ref[...] = slb

    loss_bf = loss.astype(jnp.bfloat16)
    bits = lax.bitcast_convert_type(loss_bf, jnp.int16).astype(jnp.int32)
    bin1 = bits >> 7
    onehot = (bin1 == lax.broadcasted_iota(
        jnp.int32, (_ROWS_A, 256), 1)).astype(jnp.bfloat16)
    mselb = (slb >= 0).astype(jnp.bfloat16)
    wselb = jnp.maximum(slb, jnp.bfloat16(0.0))

    dn = (((0,), (0,)), ((), ()))

    @pl.when(pl.program_id(0) == 0)
    def _():
        h1_ref[...] = jnp.zeros((_C, 256), jnp.float32)
        w1_ref[...] = jnp.zeros((_C, 256), jnp.float32)

    h1_ref[...] += lax.dot_general(mselb, onehot, dn,
                                   preferred_element_type=jnp.float32)
    w1_ref[...] += lax.dot_general(wselb, onehot, dn,
                                   preferred_element_type=jnp.float32)


def _rev_cumsum(h, nbins):
    x = h
    s = 1
    while s < nbins:
        shifted = jnp.concatenate(
            [x[:, s:], jnp.zeros((_C, s), jnp.float32)], axis=1)
        x = x + shifted
        s *= 2
    return x


def _pick_bin(t_cum, m_res, nbins):
    ge = (t_cum >= m_res).astype(jnp.float32)
    b = jnp.sum(ge, axis=1, keepdims=True) - 1.0
    return b.astype(jnp.int32)


def _lane_pick(arr, idx, nbins):
    qi = lax.broadcasted_iota(jnp.int32, (_C, nbins), 1)
    pick = qi == idx
    return jnp.sum(jnp.where(pick, arr, 0.0), axis=1, keepdims=True)


def _pass_b(sl_ref, h1_ref, w1_ref, mask_ref, out_ref, h2_ref, w2_ref):
    j = pl.program_id(0)
    nsteps = pl.num_programs(0)

    ts = jnp.sum(mask_ref[...])
    k = ts.astype(jnp.int32) // 2
    h1 = h1_ref[...]
    t1 = _rev_cumsum(h1, 256)
    cnt = t1[:, 0:1]
    m = jnp.minimum(cnt.astype(jnp.int32), k)
    m_f = m.astype(jnp.float32)
    b1 = _pick_bin(t1, m_f, 256)
    t1_b = _lane_pick(t1, b1, 256)
    h1_b = _lane_pick(h1, b1, 256)
    m2 = m_f - (t1_b - h1_b)

    ch = sl_ref[...]
    rmax = jnp.max(ch, axis=1, keepdims=True)
    bits = lax.bitcast_convert_type(rmax, jnp.int16).astype(jnp.int32)
    bin1r = bits >> 7
    bin2r = bits & 127
    inb1 = bin1r == jnp.reshape(b1, (1, _C))
    selm = ch >= 0
    keep = selm & inb1
    msel2 = keep.astype(jnp.bfloat16)
    wsel2 = jnp.where(keep, ch, jnp.bfloat16(0.0))
    oh2 = (bin2r == lax.broadcasted_iota(
        jnp.int32, (_CH, 128), 1)).astype(jnp.bfloat16)

    dn = (((0,), (0,)), ((), ()))

    @pl.when(j == 0)
    def _():
        h2_ref[...] = jnp.zeros((_C, 128), jnp.float32)
        w2_ref[...] = jnp.zeros((_C, 128), jnp.float32)

    h2_ref[...] += lax.dot_general(msel2, oh2, dn,
                                   preferred_element_type=jnp.float32)
    w2_ref[...] += lax.dot_general(wsel2, oh2, dn,
                                   preferred_element_type=jnp.float32)

    @pl.when(j == nsteps - 1)
    def _():
        h2 = h2_ref[...]
        w2 = w2_ref[...]
        t2 = _rev_cumsum(h2, 128)
        b2 = _pick_bin(t2, m2, 128)
        t2_b = _lane_pick(t2, b2, 128)
        h2_b = _lane_pick(h2, b2, 128)

        tbits = (b1 << 7) | b2
        tval = lax.bitcast_convert_type(
            tbits.astype(jnp.int16), jnp.bfloat16).astype(jnp.float32)

        w1 = w1_ref[...]
        wr1 = _rev_cumsum(w1, 256)
        w1_above = _lane_pick(wr1, b1, 256) - _lane_pick(w1, b1, 256)
        wr2 = _rev_cumsum(w2, 128)
        w2_above = _lane_pick(wr2, b2, 128) - _lane_pick(w2, b2, 128)

        s_gt = w1_above + w2_above
        c_gt = (t1_b - h1_b) + (t2_b - h2_b)

        class_sum = s_gt + (m_f - c_gt) * tval
        class_loss = class_sum / m_f
        valid = cnt > 0.0
        num_valid = jnp.sum(valid.astype(jnp.float32))
        mean_valid = jnp.sum(jnp.where(valid, class_loss, 0.0)) / num_valid
        result = jnp.where(ts > 0.0, mean_valid, 0.0)
        out_ref[...] = result * jnp.ones((1, 1), jnp.float32)


def kernel(y_pred, y_true, mask):
    n, c = y_pred.shape
    mask_f = mask.astype(jnp.float32).reshape(n, 1)

    sel_loss, h1, w1 = pl.pallas_call(
        _pass_a,
        grid=(n // _ROWS_A,),
        in_specs=[
            pl.BlockSpec((_ROWS_A, c), lambda i: (i, 0)),
            pl.BlockSpec((_ROWS_A, c), lambda i: (i, 0)),
            pl.BlockSpec((_ROWS_A, 1), lambda i: (i, 0)),
        ],
        out_specs=[
            pl.BlockSpec((_ROWS_A, c), lambda i: (i, 0)),
            pl.BlockSpec((_C, 256), lambda i: (0, 0)),
            pl.BlockSpec((_C, 256), lambda i: (0, 0)),
        ],
        out_shape=[
            jax.ShapeDtypeStruct((n, c), jnp.bfloat16),
            jax.ShapeDtypeStruct((_C, 256), jnp.float32),
            jax.ShapeDtypeStruct((_C, 256), jnp.float32),
        ],
    )(y_pred, y_true, mask_f)

    mask_rm = mask_f.reshape(n // 128, 128)

    out = pl.pallas_call(
        _pass_b,
        grid=(n // _CH,),
        in_specs=[
            pl.BlockSpec((_CH, c), lambda j: (j, 0)),
            pl.BlockSpec((_C, 256), lambda j: (0, 0)),
            pl.BlockSpec((_C, 256), lambda j: (0, 0)),
            pl.BlockSpec((n // 128, 128), lambda j: (0, 0)),
        ],
        out_specs=pl.BlockSpec((1, 1), lambda j: (0, 0)),
        out_shape=jax.ShapeDtypeStruct((1, 1), jnp.float32),
        scratch_shapes=[
            pltpu.VMEM((_C, 128), jnp.float32),
            pltpu.VMEM((_C, 128), jnp.float32),
        ],
    )(sel_loss, h1, w1, mask_rm)

    return out[0, 0]


# speedup vs baseline: 1.1868x; 1.0032x over previous
import jax
import jax.numpy as jnp
from jax import lax
from jax.experimental import pallas as pl
from jax.experimental.pallas import tpu as pltpu

_N = 65536
_C = 80
_ROWS_A = 8192
_CH = 16384


def _pass_a(yp_ref, yt_ref, mask_ref, sl_ref, h1_ref, w1_ref):
    p = yp_ref[...]
    t = yt_ref[...]
    log_p = jnp.maximum(jnp.log(p), -100.0)
    log_1p = jnp.maximum(jnp.log(1.0 - p), -100.0)
    l = -(t * log_p + (1.0 - t) * log_1p)
    loss = jnp.sum(l, axis=1, keepdims=True)
    mk = mask_ref[...]
    sel = (mk > 0.5) & (t > 0.5)
    slf = jnp.where(sel, loss, -1.0)
    slb = slf.astype(jnp.bfloat16)
    sl_ref[...] = slb

    loss_bf = loss.astype(jnp.bfloat16)
    bits = lax.bitcast_convert_type(loss_bf, jnp.int16).astype(jnp.int32)
    bin1 = bits >> 7
    onehot = (bin1 == lax.broadcasted_iota(
        jnp.int32, (_ROWS_A, 256), 1)).astype(jnp.bfloat16)
    mselb = (slb >= 0).astype(jnp.bfloat16)
    wselb = jnp.maximum(slb, jnp.bfloat16(0.0))

    dn = (((0,), (0,)), ((), ()))

    @pl.when(pl.program_id(0) == 0)
    def _():
        h1_ref[...] = jnp.zeros((_C, 256), jnp.float32)
        w1_ref[...] = jnp.zeros((_C, 256), jnp.float32)

    h1_ref[...] += lax.dot_general(mselb, onehot, dn,
                                   preferred_element_type=jnp.float32)
    w1_ref[...] += lax.dot_general(wselb, onehot, dn,
                                   preferred_element_type=jnp.float32)


def _rev_cumsum(h, nbins):
    x = h
    s = 1
    while s < nbins:
        shifted = jnp.concatenate(
            [x[:, s:], jnp.zeros((_C, s), jnp.float32)], axis=1)
        x = x + shifted
        s *= 2
    return x


def _pick_bin(t_cum, m_res, nbins):
    ge = (t_cum >= m_res).astype(jnp.float32)
    b = jnp.sum(ge, axis=1, keepdims=True) - 1.0
    return b.astype(jnp.int32)


def _lane_pick(arr, idx, nbins):
    qi = lax.broadcasted_iota(jnp.int32, (_C, nbins), 1)
    pick = qi == idx
    return jnp.sum(jnp.where(pick, arr, 0.0), axis=1, keepdims=True)


def _pass_b(sl_ref, h1_ref, w1_ref, mask_ref, out_ref, h2_ref, w2_ref):
    j = pl.program_id(0)
    nsteps = pl.num_programs(0)

    ts = jnp.sum(mask_ref[...])
    k = ts.astype(jnp.int32) // 2
    h1 = h1_ref[...]
    t1 = _rev_cumsum(h1, 256)
    cnt = t1[:, 0:1]
    m = jnp.minimum(cnt.astype(jnp.int32), k)
    m_f = m.astype(jnp.float32)
    b1 = _pick_bin(t1, m_f, 256)
    t1_b = _lane_pick(t1, b1, 256)
    h1_b = _lane_pick(h1, b1, 256)
    m2 = m_f - (t1_b - h1_b)

    ch = sl_ref[...]
    rmax = jnp.max(ch, axis=1, keepdims=True)
    bits = lax.bitcast_convert_type(rmax, jnp.int16).astype(jnp.int32)
    bin1r = bits >> 7
    bin2r = bits & 127
    inb1 = (bin1r == jnp.reshape(b1, (1, _C))).astype(jnp.bfloat16)
    selm = (ch >= 0).astype(jnp.bfloat16)
    msel2 = selm * inb1
    wsel2 = jnp.maximum(ch, jnp.bfloat16(0.0)) * inb1
    oh2 = (bin2r == lax.broadcasted_iota(
        jnp.int32, (_CH, 128), 1)).astype(jnp.bfloat16)

    dn = (((0,), (0,)), ((), ()))

    @pl.when(j == 0)
    def _():
        h2_ref[...] = jnp.zeros((_C, 128), jnp.float32)
        w2_ref[...] = jnp.zeros((_C, 128), jnp.float32)

    h2_ref[...] += lax.dot_general(msel2, oh2, dn,
                                   preferred_element_type=jnp.float32)
    w2_ref[...] += lax.dot_general(wsel2, oh2, dn,
                                   preferred_element_type=jnp.float32)

    @pl.when(j == nsteps - 1)
    def _():
        h2 = h2_ref[...]
        w2 = w2_ref[...]
        t2 = _rev_cumsum(h2, 128)
        b2 = _pick_bin(t2, m2, 128)
        t2_b = _lane_pick(t2, b2, 128)
        h2_b = _lane_pick(h2, b2, 128)

        tbits = (b1 << 7) | b2
        tval = lax.bitcast_convert_type(
            tbits.astype(jnp.int16), jnp.bfloat16).astype(jnp.float32)

        w1 = w1_ref[...]
        wr1 = _rev_cumsum(w1, 256)
        w1_above = _lane_pick(wr1, b1, 256) - _lane_pick(w1, b1, 256)
        wr2 = _rev_cumsum(w2, 128)
        w2_above = _lane_pick(wr2, b2, 128) - _lane_pick(w2, b2, 128)

        s_gt = w1_above + w2_above
        c_gt = (t1_b - h1_b) + (t2_b - h2_b)

        class_sum = s_gt + (m_f - c_gt) * tval
        class_loss = class_sum / m_f
        valid = cnt > 0.0
        num_valid = jnp.sum(valid.astype(jnp.float32))
        mean_valid = jnp.sum(jnp.where(valid, class_loss, 0.0)) / num_valid
        result = jnp.where(ts > 0.0, mean_valid, 0.0)
        out_ref[...] = result * jnp.ones((1, 1), jnp.float32)


def kernel(y_pred, y_true, mask):
    n, c = y_pred.shape
    mask_f = mask.astype(jnp.float32).reshape(n, 1)

    sel_loss, h1, w1 = pl.pallas_call(
        _pass_a,
        grid=(n // _ROWS_A,),
        in_specs=[
            pl.BlockSpec((_ROWS_A, c), lambda i: (i, 0)),
            pl.BlockSpec((_ROWS_A, c), lambda i: (i, 0)),
            pl.BlockSpec((_ROWS_A, 1), lambda i: (i, 0)),
        ],
        out_specs=[
            pl.BlockSpec((_ROWS_A, c), lambda i: (i, 0)),
            pl.BlockSpec((_C, 256), lambda i: (0, 0)),
            pl.BlockSpec((_C, 256), lambda i: (0, 0)),
        ],
        out_shape=[
            jax.ShapeDtypeStruct((n, c), jnp.bfloat16),
            jax.ShapeDtypeStruct((_C, 256), jnp.float32),
            jax.ShapeDtypeStruct((_C, 256), jnp.float32),
        ],
    )(y_pred, y_true, mask_f)

    mask_rm = mask_f.reshape(n // 128, 128)

    out = pl.pallas_call(
        _pass_b,
        grid=(n // _CH,),
        in_specs=[
            pl.BlockSpec((_CH, c), lambda j: (j, 0)),
            pl.BlockSpec((_C, 256), lambda j: (0, 0)),
            pl.BlockSpec((_C, 256), lambda j: (0, 0)),
            pl.BlockSpec((n // 128, 128), lambda j: (0, 0)),
        ],
        out_specs=pl.BlockSpec((1, 1), lambda j: (0, 0)),
        out_shape=jax.ShapeDtypeStruct((1, 1), jnp.float32),
        scratch_shapes=[
            pltpu.VMEM((_C, 128), jnp.float32),
            pltpu.VMEM((_C, 128), jnp.float32),
        ],
    )(sel_loss, h1, w1, mask_rm)

    return out[0, 0]
